# baseline (speedup 1.0000x reference)
import functools

import jax
import jax.numpy as jnp
from jax import lax
from jax.experimental import pallas as pl
from jax.experimental.pallas import tpu as pltpu
from jax.experimental.pallas import tpu_sc as plsc

_NBUF = 2


def _make_gather(V, D, B):
    info = plsc.get_sparse_core_info()
    nc, ns = info.num_cores, info.num_subcores
    nw = nc * ns
    assert B % (8 * nw * _NBUF) == 0 and D % info.num_lanes == 0
    b_per_w = B // nw
    rows_c = b_per_w // _NBUF
    mesh = plsc.VectorSubcoreMesh(core_axis_name="c", subcore_axis_name="s")

    @functools.partial(
        pl.kernel,
        mesh=mesh,
        out_type=jax.ShapeDtypeStruct((B, D), jnp.float32),
        scratch_types=[
            pltpu.VMEM((_NBUF, rows_c), jnp.int32),
            pltpu.VMEM((_NBUF, rows_c, D), jnp.float32),
            pltpu.SemaphoreType.DMA((_NBUF,)),
            pltpu.SemaphoreType.DMA((_NBUF,)),
            pltpu.SemaphoreType.DMA,
        ],
    )
    def gather_kernel(idx_hbm, table_hbm, out_hbm, idx_v, rows_v, isems, gsems, wsem):
        wid = lax.axis_index("s") * nc + lax.axis_index("c")
        base = wid * b_per_w
        idx_copies = [
            pltpu.async_copy(idx_hbm.at[wid, c], idx_v.at[c], isems.at[c])
            for c in range(_NBUF)
        ]
        gathers = []
        for c in range(_NBUF):
            idx_copies[c].wait()
            gathers.append(
                pltpu.async_copy(table_hbm.at[idx_v.at[c]], rows_v.at[c], gsems.at[c])
            )
        writes = []
        for c in range(_NBUF):
            gathers[c].wait()
            writes.append(
                pltpu.async_copy(
                    rows_v.at[c], out_hbm.at[pl.ds(base + c * rows_c, rows_c)], wsem
                )
            )
        for w in writes:
            w.wait()

    return gather_kernel


def kernel(class_ids, prototypes):
    V, D = prototypes.shape
    (B,) = class_ids.shape
    info = plsc.get_sparse_core_info()
    nw = info.num_cores * info.num_subcores
    gather = _make_gather(V, D, B)
    idx = class_ids.astype(jnp.int32).reshape(nw, _NBUF, B // (nw * _NBUF))
    return gather(idx, prototypes)
